# Initial kernel scaffold; baseline (speedup 1.0000x reference)
import functools

import jax
import jax.numpy as jnp
from jax import lax
from jax.experimental import pallas as pl
from jax.experimental.pallas import tpu as pltpu
from jax.experimental.pallas import tpu_sc as plsc

F32 = jnp.float32

_B = 16384
_F = 26
_D = 16
_NC = 2
_NS = 16
_NW = _NC * _NS
_PW = _B // _NW
_C = 128
_NK = _PW // _C
_G = _C // 16


@functools.partial(
    pl.kernel,
    out_type=jax.ShapeDtypeStruct((_B,), F32),
    mesh=plsc.VectorSubcoreMesh(core_axis_name="c", subcore_axis_name="s"),
    scratch_types=[
        pltpu.VMEM((_F, _C), jnp.int32),
        pltpu.VMEM((_F, _C, _D), F32),
        pltpu.VMEM((_F, _C), F32),
        pltpu.VMEM((16,), F32),
        pltpu.VMEM((_PW,), F32),
        pltpu.SemaphoreType.DMA,
    ],
)
def _fm_sc(xg, emb, lin, bias16, out, idx_v, rows_v, lin_v, bias_v, out_v, sem):
    wid = lax.axis_index("s") * _NC + lax.axis_index("c")
    base = wid * _PW

    pltpu.sync_copy(bias16, bias_v)
    bias_vec = bias_v[...]

    for k in range(_NK):
        pltpu.sync_copy(xg.at[wid, k], idx_v)
        cps = []
        for f in range(_F):
            cps.append(pltpu.async_copy(emb.at[idx_v.at[f]], rows_v.at[f], sem))
            cps.append(pltpu.async_copy(lin.at[idx_v.at[f]], lin_v.at[f], sem))
        for cp in cps:
            cp.wait()

        def group_body(g, carry):
            row0 = g * 16
            rows16 = row0 + lax.iota(jnp.int32, 16)
            lin_acc = jnp.zeros((16,), F32)
            for f in range(_F):
                lin_acc = lin_acc + lin_v[f, pl.ds(row0, 16)]
            inter = jnp.zeros((16,), F32)
            for d in range(_D):
                sd = jnp.zeros((16,), F32)
                ssd = jnp.zeros((16,), F32)
                d_splat = jnp.full((16,), d, jnp.int32)
                for f in range(_F):
                    f_splat = jnp.full((16,), f, jnp.int32)
                    e = plsc.load_gather(rows_v, [f_splat, rows16, d_splat])
                    sd = sd + e
                    ssd = ssd + e * e
                inter = inter + (sd * sd - ssd)
            z = lin_acc + bias_vec + 0.5 * inter
            out_v[pl.ds(k * _C + row0, 16)] = 1.0 / (1.0 + jnp.exp(-z))
            return carry

        lax.fori_loop(0, _G, group_body, 0)

    pltpu.sync_copy(out_v, out.at[pl.ds(base, _PW)])


def kernel(x, emb_table, linear_table, bias):
    xg = x.astype(jnp.int32).reshape(_NW, _NK, _C, _F).transpose(0, 1, 3, 2)
    lin_flat = linear_table.reshape(-1).astype(F32)
    bias16 = jnp.broadcast_to(bias.astype(F32), (16,))
    out = _fm_sc(xg, emb_table.astype(F32), lin_flat, bias16)
    return out.reshape(_B, 1)


# speedup vs baseline: 1.2588x; 1.2588x over previous
import functools

import jax
import jax.numpy as jnp
from jax import lax
from jax.experimental import pallas as pl
from jax.experimental.pallas import tpu as pltpu
from jax.experimental.pallas import tpu_sc as plsc

F32 = jnp.float32

_B = 16384
_F = 26
_D = 16
_NC = 2
_NS = 16
_NW = _NC * _NS
_PW = _B // _NW
_C = 128
_NK = _PW // _C
_G = _C // 16


@functools.partial(
    pl.kernel,
    out_type=jax.ShapeDtypeStruct((_B,), F32),
    mesh=plsc.VectorSubcoreMesh(core_axis_name="c", subcore_axis_name="s"),
    compiler_params=pltpu.CompilerParams(
        needs_layout_passes=False, use_tc_tiling_on_sc=False
    ),
    scratch_types=[
        pltpu.VMEM((_F, _C), jnp.int32),
        pltpu.VMEM((_F, _C, _D), F32),
        pltpu.VMEM((_F, _C), F32),
        pltpu.VMEM((16,), F32),
        pltpu.VMEM((_PW,), F32),
        pltpu.SemaphoreType.DMA,
    ],
)
def _fm_sc(xg, emb, lin, bias16, out, idx_v, rows_v, lin_v, bias_v, out_v, sem):
    wid = lax.axis_index("s") * _NC + lax.axis_index("c")
    base = wid * _PW

    pltpu.sync_copy(bias16, bias_v)
    bias_vec = bias_v[...]

    for k in range(_NK):
        pltpu.sync_copy(xg.at[wid, k], idx_v)
        cps = []
        for f in range(_F):
            cps.append(pltpu.async_copy(emb.at[idx_v.at[f]], rows_v.at[f], sem))
            cps.append(pltpu.async_copy(lin.at[idx_v.at[f]], lin_v.at[f], sem))
        for cp in cps:
            cp.wait()

        def group_body(g, carry):
            row0 = g * 16
            rows16 = row0 + lax.iota(jnp.int32, 16)
            lin_acc = jnp.zeros((16,), F32)
            for f in range(_F):
                lin_acc = lin_acc + lin_v[f, pl.ds(row0, 16)]
            inter = jnp.zeros((16,), F32)
            for d in range(_D):
                sd = jnp.zeros((16,), F32)
                ssd = jnp.zeros((16,), F32)
                d_splat = jnp.full((16,), d, jnp.int32)
                for f in range(_F):
                    f_splat = jnp.full((16,), f, jnp.int32)
                    e = plsc.load_gather(rows_v, [f_splat, rows16, d_splat])
                    sd = sd + e
                    ssd = ssd + e * e
                inter = inter + (sd * sd - ssd)
            z = lin_acc + bias_vec + 0.5 * inter
            out_v[pl.ds(k * _C + row0, 16)] = 1.0 / (1.0 + jnp.exp(-z))
            return carry

        lax.fori_loop(0, _G, group_body, 0)

    pltpu.sync_copy(out_v, out.at[pl.ds(base, _PW)])


def kernel(x, emb_table, linear_table, bias):
    xg = x.astype(jnp.int32).reshape(_NW, _NK, _C, _F).transpose(0, 1, 3, 2)
    lin_flat = linear_table.reshape(-1).astype(F32)
    bias16 = jnp.broadcast_to(bias.astype(F32), (16,))
    out = _fm_sc(xg, emb_table.astype(F32), lin_flat, bias16)
    return out.reshape(_B, 1)
